# Initial kernel scaffold; baseline (speedup 1.0000x reference)
import jax
import jax.numpy as jnp
from jax.experimental import pallas as pl


def kernel(x, edge_index, batch, W1, as1, ad1, b1, W2, as2, ad2, b2, W3, as3, ad3, b3, Wl, bl):
    raise NotImplementedError("write your pallas kernel here")


# speedup vs baseline: 14.5471x; 14.5471x over previous
import functools

import jax
import jax.numpy as jnp
from jax import lax
from jax.experimental import pallas as pl
from jax.experimental.pallas import tpu as pltpu
from jax.experimental.pallas import tpu_sc as plsc

N = 10000
E = 160000
E_REAL = E + N
E_TOT = 172032
F_IN = 256
DH = 256
HEADS = 4
HID = 1024
NUM_GRAPHS = 64
NUM_CLASSES = 32

EPT = E_TOT // 32
EPT_PAD = 5376
ITER_A = EPT_PAD // 16
ETS = E_TOT // 16
KB = 128
NB = ETS // KB
WCH = 32
NPASS = 16

BV = 1000
NBV = N // BV


def _tc_layer_body(first, x_ref, den_ref, b_ref, w_ref, at_ref,
                   h8_ref, asad_ref, m_ref):
    i = pl.program_id(0)
    if first:
        x_eff = x_ref[...]
    else:
        agg = x_ref[...]
        den = den_ref[...]
        rec = 1.0 / (den[0] + den[1] + 1e-16)
        parts = []
        for q in range(8):
            h = q // 2
            parts.append(agg[q] * rec[:, h:h + 1])
        x_eff = jnp.concatenate(parts, axis=1) + b_ref[...]
        x_eff = jnp.maximum(x_eff, 0.0)

    asad = jnp.zeros((8, BV), dtype=jnp.float32)
    for j in range(8):
        wj = w_ref[pl.ds(j * 128, 128), :]
        hj = lax.dot_general(x_eff, wj, (((1,), (1,)), ((), ())),
                             preferred_element_type=jnp.float32)
        h8_ref[j, :, :] = hj
        atj = at_ref[:, pl.ds(j * 128, 128)]
        asad = asad + lax.dot_general(atj, hj, (((1,), (1,)), ((), ())),
                                      preferred_element_type=jnp.float32)
    asad_ref[0, :, :] = asad

    @pl.when(i == 0)
    def _():
        m_ref[...] = jnp.full((8, 128), -3e38, dtype=jnp.float32)
    mrow = jnp.max(asad, axis=1, keepdims=True)
    m_ref[...] = jnp.maximum(m_ref[...], jnp.broadcast_to(mrow, (8, 128)))


def _tc_layer(x, den, b, W, AT, first):
    if first:
        f = x.shape[1]
        in_specs = [pl.BlockSpec((BV, f), lambda i: (i, 0))]
    else:
        f = HID
        in_specs = [pl.BlockSpec((8, BV, 128), lambda i: (0, i, 0))]
    args = [x]
    if not first:
        in_specs.append(pl.BlockSpec((2, BV, HEADS), lambda i: (0, i, 0)))
        in_specs.append(pl.BlockSpec((1, HID), lambda i: (0, 0)))
        args += [den, b.reshape(1, HID)]
    in_specs.append(pl.BlockSpec((HID, f), lambda i: (0, 0)))
    in_specs.append(pl.BlockSpec((8, HID), lambda i: (0, 0)))
    args += [W, AT]

    body = functools.partial(_tc_layer_body, first)
    if first:
        def body2(x_ref, w_ref, at_ref, h8_ref, asad_ref, m_ref):
            body(x_ref, None, None, w_ref, at_ref, h8_ref, asad_ref, m_ref)
        fn = body2
    else:
        fn = body

    return pl.pallas_call(
        fn,
        grid=(NBV,),
        in_specs=in_specs,
        out_specs=[
            pl.BlockSpec((8, BV, 128), lambda i: (0, i, 0)),
            pl.BlockSpec((1, 8, BV), lambda i: (i, 0, 0)),
            pl.BlockSpec((8, 128), lambda i: (0, 0)),
        ],
        out_shape=[
            jax.ShapeDtypeStruct((8, N, 128), jnp.float32),
            jax.ShapeDtypeStruct((NBV, 8, BV), jnp.float32),
            jax.ShapeDtypeStruct((8, 128), jnp.float32),
        ],
    )(*args)


_sc_mesh = plsc.VectorSubcoreMesh(core_axis_name="c", subcore_axis_name="s")
_sc_params = pltpu.CompilerParams(needs_layout_passes=False)


@functools.partial(
    pl.kernel,
    mesh=_sc_mesh,
    compiler_params=_sc_params,
    out_type=[
        jax.ShapeDtypeStruct((HEADS * E_TOT,), jnp.float32),
        jax.ShapeDtypeStruct((2 * N * HEADS,), jnp.float32),
    ],

    scratch_types=[
        pltpu.VMEM((EPT_PAD,), jnp.int32),
        pltpu.VMEM((EPT_PAD,), jnp.int32),
        pltpu.VMEM((N,), jnp.float32),
        pltpu.VMEM((N,), jnp.float32),
        pltpu.VMEM((8, 128), jnp.float32),
        pltpu.VMEM((EPT_PAD,), jnp.float32),
        pltpu.VMEM((42, 128), jnp.int32),
        pltpu.VMEM_SHARED((N * HEADS,), jnp.float32),
    ],
)
def _sc_edge_kernel(src_hbm, dst_hbm, asad_hbm, m_hbm,
                    ex_hbm, den_hbm,
                    src_v, dst_v, as_v, ad_v, m_v, ex_v, idx_v, den_sh):
    c = lax.axis_index("c")
    s = lax.axis_index("s")
    wid = s * 2 + c
    base = wid * EPT

    def zfill(k, _):
        ex_v[pl.ds(k * 16, 16)] = jnp.zeros((16,), jnp.float32)
        return 0
    lax.fori_loop(0, ITER_A, zfill, 0)

    @pl.when(s < 8)
    def _():
        pltpu.sync_copy(ex_v.at[pl.ds(0, 5000)],
                        den_sh.at[pl.ds(s * 5000, 5000)])

    pltpu.sync_copy(src_hbm.at[pl.ds(base, EPT)], src_v.at[pl.ds(0, EPT)])
    pltpu.sync_copy(dst_hbm.at[pl.ds(base, EPT)], dst_v.at[pl.ds(0, EPT)])
    for k in range(EPT, EPT_PAD, 16):
        src_v[pl.ds(k, 16)] = jnp.zeros((16,), jnp.int32)
        dst_v[pl.ds(k, 16)] = jnp.zeros((16,), jnp.int32)
    pltpu.sync_copy(m_hbm, m_v)
    plsc.subcore_barrier()

    iota16 = lax.broadcasted_iota(jnp.int32, (16,), 0)
    for h in range(HEADS):
        for i in range(NBV):
            pltpu.sync_copy(asad_hbm.at[pl.ds((i * 8 + h) * BV, BV)],
                            as_v.at[pl.ds(i * BV, BV)])
            pltpu.sync_copy(asad_hbm.at[pl.ds((i * 8 + 4 + h) * BV, BV)],
                            ad_v.at[pl.ds(i * BV, BV)])
        msum = m_v[h, pl.ds(0, 16)][0] + m_v[4 + h, pl.ds(0, 16)][0]
        m_s = jnp.where(msum >= 0.0, msum, 0.2 * msum)

        def body(k, _):
            off = k * 16
            sidx = src_v[pl.ds(off, 16)]
            didx = dst_v[pl.ds(off, 16)]
            a = plsc.load_gather(as_v, [sidx])
            d = plsc.load_gather(ad_v, [didx])
            e = a + d
            e = jnp.where(e >= 0.0, e, 0.2 * e)
            ex = jnp.exp(e - m_s)
            lid = off + iota16
            valid = (lid < EPT) & ((base + lid) < E_REAL)
            ex = jnp.where(valid, ex, 0.0)
            ex_v[pl.ds(off, 16)] = ex
            row = k // 8
            col = (k % 8) * 16
            idx_v[row, pl.ds(col, 16)] = didx * 4 + h
            return 0

        lax.fori_loop(0, ITER_A, body, 0)

        pltpu.sync_copy(ex_v.at[pl.ds(0, EPT)],
                        ex_hbm.at[pl.ds(h * E_TOT + base, EPT)])
        for j in range(42):
            pltpu.sync_copy(ex_v.at[pl.ds(j * 128, 128)],
                            den_sh.at[idx_v.at[j]], add=True)

    plsc.subcore_barrier()
    @pl.when(s < 8)
    def _():
        pltpu.sync_copy(den_sh.at[pl.ds(s * 5000, 5000)],
                        ex_v.at[pl.ds(0, 5000)])
        pltpu.sync_copy(ex_v.at[pl.ds(0, 5000)],
                        den_hbm.at[pl.ds(c * N * HEADS + s * 5000, 5000)])


@functools.partial(
    pl.kernel,
    mesh=_sc_mesh,
    compiler_params=_sc_params,
    out_type=jax.ShapeDtypeStruct((8, N, 128), jnp.float32),
    scratch_types=[
        pltpu.VMEM((NB, KB), jnp.int32),
        pltpu.VMEM((NB, KB), jnp.int32),
        pltpu.VMEM((ETS,), jnp.float32),
        pltpu.VMEM((KB, 128), jnp.float32),
        pltpu.VMEM_SHARED((N, 128), jnp.float32),
        pltpu.SemaphoreType.DMA,
    ],
)
def _sc_agg_kernel(src2d_hbm, dst2d_hbm, ex_hbm, h8_hbm,
                   agg_hbm,
                   dst_v, gidx_v, ex_v, rows_v, slab, sem):
    c = lax.axis_index("c")
    s = lax.axis_index("s")

    pltpu.sync_copy(dst2d_hbm.at[s], dst_v)

    def one_pass(p, _):
        chunk = c * 4 + p
        head = chunk // 2

        def zrow(r, _):
            for jj in range(8):
                rows_v[r, pl.ds(jj * 16, 16)] = jnp.zeros((16,), jnp.float32)
            return 0
        lax.fori_loop(0, KB, zrow, 0)

        @pl.when(s < 10)
        def _():
            def zslab(k, _):
                pltpu.sync_copy(rows_v.at[pl.ds(0, 40)],
                                slab.at[pl.ds(s * 1000 + k * 40, 40)])
                return 0
            lax.fori_loop(0, 25, zslab, 0)

        pltpu.sync_copy(ex_hbm.at[pl.ds(head * E_TOT + s * ETS, ETS)], ex_v)
        pltpu.sync_copy(src2d_hbm.at[s], gidx_v)
        off = chunk * N

        def gbody(k, _):
            r = k // 8
            col = (k % 8) * 16
            gidx_v[r, pl.ds(col, 16)] = gidx_v[r, pl.ds(col, 16)] + off
            return 0
        lax.fori_loop(0, NB * 8, gbody, 0)
        plsc.subcore_barrier()

        def block(blk, _):
            pltpu.async_copy(h8_hbm.at[gidx_v.at[blk]], rows_v, sem).wait()

            def scale(b2, _):
                exv = ex_v[pl.ds(blk * KB + b2 * 16, 16)]
                for t in range(16):
                    exs = exv[t]
                    e2 = b2 * 16 + t
                    for j in range(8):
                        sl = pl.ds(j * 16, 16)
                        rows_v[e2, sl] = rows_v[e2, sl] * exs
                return 0
            lax.fori_loop(0, KB // 16, scale, 0)
            pltpu.sync_copy(rows_v, slab.at[dst_v.at[blk]], add=True)
            return 0

        lax.fori_loop(0, NB, block, 0)
        plsc.subcore_barrier()

        @pl.when(s < 10)
        def _():
            def wb(k, _):
                r0 = s * 1000 + k * 40
                pltpu.sync_copy(slab.at[pl.ds(r0, 40)],
                                rows_v.at[pl.ds(0, 40)])
                pltpu.sync_copy(rows_v.at[pl.ds(0, 40)],
                                agg_hbm.at[chunk, pl.ds(r0, 40)])
                return 0
            lax.fori_loop(0, 25, wb, 0)
        plsc.subcore_barrier()
        return 0

    lax.fori_loop(0, 4, one_pass, 0)


def _tc_pool_body(agg_ref, den_ref, b_ref, batch_ref, wl_ref, bl_ref,
                  out_ref, g_ref):
    i = pl.program_id(0)
    agg = agg_ref[...]
    den = den_ref[...]
    rec = 1.0 / (den[0] + den[1] + 1e-16)
    parts = []
    for q in range(8):
        h = q // 2
        parts.append(agg[q] * rec[:, h:h + 1])
    x3 = jnp.concatenate(parts, axis=1) + b_ref[...]

    bb = batch_ref[0, 0, :]
    gi = lax.broadcasted_iota(jnp.int32, (NUM_GRAPHS, BV), 0)
    oh = jnp.where(gi == bb[None, :], 1.0, 0.0)

    @pl.when(i == 0)
    def _():
        g_ref[...] = jnp.zeros((NUM_GRAPHS, HID), jnp.float32)
    g_ref[...] = g_ref[...] + jnp.dot(oh, x3,
                                      preferred_element_type=jnp.float32)

    @pl.when(i == NBV - 1)
    def _():
        out_ref[...] = lax.dot_general(
            g_ref[...], wl_ref[...], (((1,), (1,)), ((), ())),
            preferred_element_type=jnp.float32) + bl_ref[...]


def _tc_pool(agg, den, b, batch3, Wl, bl):
    return pl.pallas_call(
        _tc_pool_body,
        grid=(NBV,),
        in_specs=[
            pl.BlockSpec((8, BV, 128), lambda i: (0, i, 0)),
            pl.BlockSpec((2, BV, HEADS), lambda i: (0, i, 0)),
            pl.BlockSpec((1, HID), lambda i: (0, 0)),
            pl.BlockSpec((1, 1, BV), lambda i: (i, 0, 0)),
            pl.BlockSpec((NUM_CLASSES, HID), lambda i: (0, 0)),
            pl.BlockSpec((1, NUM_CLASSES), lambda i: (0, 0)),
        ],
        out_specs=pl.BlockSpec((NUM_GRAPHS, NUM_CLASSES), lambda i: (0, 0)),
        out_shape=jax.ShapeDtypeStruct((NUM_GRAPHS, NUM_CLASSES), jnp.float32),
        scratch_shapes=[pltpu.VMEM((NUM_GRAPHS, HID), jnp.float32)],
    )(agg, den, b.reshape(1, HID), batch3, Wl, bl.reshape(1, NUM_CLASSES))


def _make_at(a_s, a_d):
    z = jnp.zeros((HEADS, HEADS, DH), jnp.float32)
    r = jnp.arange(HEADS)
    bs = z.at[r, r].set(a_s).reshape(HEADS, HID)
    bd = z.at[r, r].set(a_d).reshape(HEADS, HID)
    return jnp.concatenate([bs, bd], axis=0)


def kernel(x, edge_index, batch, W1, as1, ad1, b1, W2, as2, ad2, b2,
           W3, as3, ad3, b3, Wl, bl):
    x = x.astype(jnp.float32)
    ei = edge_index.astype(jnp.int32)
    loop = jnp.arange(N, dtype=jnp.int32)
    pad = jnp.zeros((E_TOT - E_REAL,), jnp.int32)
    src = jnp.concatenate([ei[0], loop, pad])
    dst = jnp.concatenate([ei[1], loop, pad])
    src2d = src.reshape(16, NB, KB)
    dst2d = dst.reshape(16, NB, KB)
    batch3 = batch.astype(jnp.int32).reshape(NBV, 1, BV)

    agg, den = None, None
    layers = [(W1, as1, ad1, None), (W2, as2, ad2, b1), (W3, as3, ad3, b2)]
    for li, (W, a_s, a_d, bprev) in enumerate(layers):
        AT = _make_at(a_s, a_d)
        first = li == 0
        xin = x if first else agg
        h8, asad, m8 = _tc_layer(xin, den, bprev, W, AT, first)
        ex, den = _sc_edge_kernel(src, dst, asad.reshape(-1), m8)
        den = den.reshape(2, N, HEADS)
        h8flat = h8.reshape(8 * N, 128)
        agg = _sc_agg_kernel(src2d, dst2d, ex, h8flat)

    return _tc_pool(agg, den, b3, batch3, Wl, bl)
